# baseline (speedup 1.0000x reference)
import functools

import jax
import jax.numpy as jnp
from jax import lax
from jax.experimental import pallas as pl
from jax.experimental.pallas import tpu as pltpu
from jax.experimental.pallas import tpu_sc as plsc

NUM_CORES = 2
NUM_SUBCORES = 16
NUM_WORKERS = NUM_CORES * NUM_SUBCORES
LANES = 16
ROWS = 64
COLS = 32768
SC_ROWS = 32
TC_BLOCK_ROWS = 8
TC_GRID = (ROWS - SC_ROWS) // TC_BLOCK_ROWS
VECS_PER_ROW = COLS // LANES

_MESH = plsc.VectorSubcoreMesh(
    core_axis_name="c",
    subcore_axis_name="s",
    num_cores=NUM_CORES,
    num_subcores=NUM_SUBCORES,
)


@functools.partial(
    pl.kernel,
    out_type=jax.ShapeDtypeStruct((NUM_WORKERS, LANES), jnp.float32),
    mesh=_MESH,
    scratch_types=[
        pltpu.VMEM((COLS + LANES,), jnp.float32),
        pltpu.VMEM((LANES,), jnp.float32),
        pltpu.SemaphoreType.DMA,
    ],
)
def _sc_check(x_hbm, out_hbm, buf, res_v, sem):
    wid = lax.axis_index("s") * NUM_CORES + lax.axis_index("c")
    cp = pltpu.async_copy(
        x_hbm.at[pl.ds(wid * COLS, COLS)], buf.at[pl.ds(0, COLS)], sem
    )
    cp.wait()
    buf[pl.ds(COLS, LANES)] = jnp.full((LANES,), jnp.inf, dtype=jnp.float32)

    def body(i, acc):
        j = i * LANES
        a = buf[pl.ds(j, LANES)]
        b = buf[pl.ds(j + 1, LANES)]
        return acc + jnp.where(a > b, 1.0, 0.0)

    acc = lax.fori_loop(
        0, VECS_PER_ROW, body, jnp.zeros((LANES,), jnp.float32), unroll=4
    )
    res_v[...] = acc
    pltpu.sync_copy(res_v, out_hbm.at[wid])


def _tc_body(x_ref, o_ref):
    i = pl.program_id(0)
    blk = x_ref[...]
    viol = jnp.sum((blk[:, :-1] > blk[:, 1:]).astype(jnp.float32))
    o_ref[pl.ds(i, 1), :] = jnp.full((1, 128), viol, dtype=jnp.float32)


_tc_check = pl.pallas_call(
    _tc_body,
    grid=(TC_GRID,),
    in_specs=[
        pl.BlockSpec((TC_BLOCK_ROWS, COLS), lambda i: (i, 0)),
    ],
    out_specs=pl.BlockSpec((TC_GRID, 128), lambda i: (0, 0)),
    out_shape=jax.ShapeDtypeStruct((TC_GRID, 128), jnp.float32),
)


def kernel(x):
    xf = x.reshape(-1)
    sc_counts = _sc_check(xf[: SC_ROWS * COLS])
    tc_counts = _tc_check(x[SC_ROWS:])
    total = jnp.sum(sc_counts) + jnp.sum(tc_counts[:, 0])
    return (total == 0.0).astype(jnp.float32)


# speedup vs baseline: 1.0007x; 1.0007x over previous
import functools

import jax
import jax.numpy as jnp
from jax import lax
from jax.experimental import pallas as pl
from jax.experimental.pallas import tpu as pltpu
from jax.experimental.pallas import tpu_sc as plsc

NUM_CORES = 2
NUM_SUBCORES = 16
NUM_WORKERS = NUM_CORES * NUM_SUBCORES
LANES = 16
ROWS = 64
COLS = 32768
SC_ROWS = 32
TC_BLOCK_ROWS = 8
TC_GRID = (ROWS - SC_ROWS) // TC_BLOCK_ROWS
VECS_PER_ROW = COLS // LANES

_MESH = plsc.VectorSubcoreMesh(
    core_axis_name="c",
    subcore_axis_name="s",
    num_cores=NUM_CORES,
    num_subcores=NUM_SUBCORES,
)


@functools.partial(
    pl.kernel,
    out_type=jax.ShapeDtypeStruct((NUM_WORKERS, LANES), jnp.float32),
    mesh=_MESH,
    scratch_types=[
        pltpu.VMEM((COLS + LANES,), jnp.float32),
        pltpu.VMEM((LANES,), jnp.float32),
        pltpu.SemaphoreType.DMA,
    ],
)
def _sc_check(x_hbm, out_hbm, buf, res_v, sem):
    wid = lax.axis_index("s") * NUM_CORES + lax.axis_index("c")
    cp = pltpu.async_copy(
        x_hbm.at[pl.ds(wid * COLS, COLS)], buf.at[pl.ds(0, COLS)], sem
    )
    cp.wait()
    buf[pl.ds(COLS, LANES)] = jnp.full((LANES,), jnp.inf, dtype=jnp.float32)

    def body(i, acc):
        j = i * LANES
        a = buf[pl.ds(j, LANES)]
        b = buf[pl.ds(j + 1, LANES)]
        return acc + jnp.where(a > b, 1.0, 0.0)

    acc = lax.fori_loop(
        0, VECS_PER_ROW, body, jnp.zeros((LANES,), jnp.float32), unroll=4
    )
    res_v[...] = acc
    pltpu.sync_copy(res_v, out_hbm.at[wid])


def _tc_body(x_ref, o_ref):
    i = pl.program_id(0)
    blk = x_ref[...]
    viol = jnp.sum((blk[:, :-1] > blk[:, 1:]).astype(jnp.float32))
    o_ref[pl.ds(i, 1), :] = jnp.full((1, 128), viol, dtype=jnp.float32)


_tc_check = pl.pallas_call(
    _tc_body,
    grid=(TC_GRID,),
    in_specs=[
        pl.BlockSpec((TC_BLOCK_ROWS, COLS), lambda i: (i, 0)),
    ],
    out_specs=pl.BlockSpec((TC_GRID, 128), lambda i: (0, 0)),
    out_shape=jax.ShapeDtypeStruct((TC_GRID, 128), jnp.float32),
)


def kernel(x):
    xf = x.reshape(-1)
    tc_counts = _tc_check(x[SC_ROWS:])
    sc_counts = _sc_check(xf[: SC_ROWS * COLS])
    total = jnp.sum(sc_counts) + jnp.sum(tc_counts[:, 0])
    return (total == 0.0).astype(jnp.float32)
